# Initial kernel scaffold; baseline (speedup 1.0000x reference)
import jax
import jax.numpy as jnp
from jax.experimental import pallas as pl


def kernel(query, reference_points, input_features, input_spatial_shapes, W_off, b_off, W_attn, b_attn, W_val, b_val, W_out, b_out):
    raise NotImplementedError("write your pallas kernel here")


# speedup vs baseline: 9.0117x; 9.0117x over previous
import functools

import jax
import jax.numpy as jnp
from jax import lax
from jax.experimental import pallas as pl
from jax.experimental.pallas import tpu as pltpu
from jax.experimental.pallas import tpu_sc as plsc

_B = 4
_LQ = 4096
_DIM = 1024
_NH = 16
_HD = 64
_H = 64
_W = 64
_HW = _H * _W

_NC = 2
_NS = 16
_NW = _NC * _NS

_ROWS = _B * _LQ * _NH
_RPW = _ROWS // _NW
_CHUNK = 128
_GRP = 8
_NCH = _RPW // _CHUNK


def _vproj_body(a_ref, w_ref, b_ref, o_ref):
    o_ref[...] = (
        jnp.dot(a_ref[...], w_ref[...], preferred_element_type=jnp.float32)
        + b_ref[...]
    )


def _vproj(feat2d, w64, b64):
    m_blk = 1024
    grid = (feat2d.shape[0] // m_blk,)
    return pl.pallas_call(
        _vproj_body,
        grid=grid,
        in_specs=[
            pl.BlockSpec((m_blk, _DIM), lambda i: (i, 0)),
            pl.BlockSpec((_DIM, _HD), lambda i: (0, 0)),
            pl.BlockSpec((1, _HD), lambda i: (0, 0)),
        ],
        out_specs=pl.BlockSpec((m_blk, _HD), lambda i: (i, 0)),
        out_shape=jax.ShapeDtypeStruct((feat2d.shape[0], _HD), jnp.float32),
    )(feat2d, w64, b64)


def _index_body(q_ref, w_ref, b_ref, r_ref, o_ref, *, m_blk):
    off = (
        jnp.dot(q_ref[...], w_ref[...], preferred_element_type=jnp.float32)
        + b_ref[...]
    )
    sp = jnp.clip(r_ref[...] + off, 0.0, 1.0)
    s = sp * jnp.float32(_W - 1)
    fl = jnp.floor(s).astype(jnp.int32)
    x0 = fl[:, :_NH]
    y0 = fl[:, _NH:]
    idx = jnp.clip(y0 * _W + x0, 0, _H - 1)
    k_iota = lax.broadcasted_iota(jnp.int32, (m_blk, _NH), 1)
    b_idx = (pl.program_id(0) * m_blk) // _LQ
    o_ref[...] = idx * _W + k_iota + b_idx * _HW


def _indices(q2d, w_offc, b_offc, refxy):
    m_blk = 2048
    grid = (q2d.shape[0] // m_blk,)
    return pl.pallas_call(
        functools.partial(_index_body, m_blk=m_blk),
        grid=grid,
        in_specs=[
            pl.BlockSpec((m_blk, _DIM), lambda i: (i, 0)),
            pl.BlockSpec((_DIM, 2 * _NH), lambda i: (0, 0)),
            pl.BlockSpec((1, 2 * _NH), lambda i: (0, 0)),
            pl.BlockSpec((m_blk, 2 * _NH), lambda i: (i, 0)),
        ],
        out_specs=pl.BlockSpec((m_blk, _NH), lambda i: (i, 0)),
        out_shape=jax.ShapeDtypeStruct((q2d.shape[0], _NH), jnp.int32),
    )(q2d, w_offc, b_offc, refxy)


def _sc_gather_body(table_hbm, idx_hbm, out_hbm, idx_v, rows_v, sem):
    wid = lax.axis_index("s") * _NC + lax.axis_index("c")
    pltpu.sync_copy(idx_hbm.at[wid], idx_v)
    base = wid * _RPW
    for g in range(_NCH // _GRP):
        copies = []
        for j in range(_GRP):
            copies.append(
                pltpu.async_copy(
                    table_hbm.at[idx_v.at[g * _GRP + j]],
                    rows_v.at[pl.ds(j * _CHUNK, _CHUNK)],
                    sem,
                )
            )
        for c in copies:
            c.wait()
        pltpu.sync_copy(
            rows_v,
            out_hbm.at[pl.ds(base + g * _GRP * _CHUNK, _GRP * _CHUNK)],
        )


def _sc_gather(table, idx3d):
    mesh = plsc.VectorSubcoreMesh(core_axis_name="c", subcore_axis_name="s")
    fn = pl.kernel(
        _sc_gather_body,
        out_type=jax.ShapeDtypeStruct((_ROWS, _HD), jnp.float32),
        mesh=mesh,
        compiler_params=pltpu.CompilerParams(use_tc_tiling_on_sc=False),
        scratch_types=[
            pltpu.VMEM((_NCH, _CHUNK), jnp.int32),
            pltpu.VMEM((_GRP * _CHUNK, _HD), jnp.float32),
            pltpu.SemaphoreType.DMA,
        ],
    )
    return fn(table, idx3d)


def _outproj_body(a_ref, w_ref, b_ref, o_ref):
    o_ref[...] = (
        jnp.dot(a_ref[...], w_ref[...], preferred_element_type=jnp.float32)
        + b_ref[...]
    )


def _outproj(a2d, w, b):
    m_blk = 512
    grid = (a2d.shape[0] // m_blk,)
    return pl.pallas_call(
        _outproj_body,
        grid=grid,
        in_specs=[
            pl.BlockSpec((m_blk, _DIM), lambda i: (i, 0)),
            pl.BlockSpec((_DIM, _DIM), lambda i: (0, 0)),
            pl.BlockSpec((1, _DIM), lambda i: (0, 0)),
        ],
        out_specs=pl.BlockSpec((m_blk, _DIM), lambda i: (i, 0)),
        out_shape=jax.ShapeDtypeStruct((a2d.shape[0], _DIM), jnp.float32),
    )(a2d, w, b)


def kernel(query, reference_points, input_features, input_spatial_shapes,
           W_off, b_off, W_attn, b_attn, W_val, b_val, W_out, b_out):
    del input_spatial_shapes, W_attn, b_attn

    feat2d = input_features.reshape(_B * _HW, _DIM)
    q2d = query.reshape(_B * _LQ, _DIM)

    w_offc = jnp.concatenate([W_off[:, 0::2], W_off[:, 1::2]], axis=1)
    b_offc = jnp.concatenate([b_off[0::2], b_off[1::2]]).reshape(1, 2 * _NH)
    rp = reference_points[:, :, 0, :].reshape(_B * _LQ, 2)
    refxy = jnp.concatenate(
        [
            jnp.broadcast_to(rp[:, 0:1], (_B * _LQ, _NH)),
            jnp.broadcast_to(rp[:, 1:2], (_B * _LQ, _NH)),
        ],
        axis=1,
    )

    table = _vproj(feat2d, W_val[:, :_HD], b_val[:_HD].reshape(1, _HD))
    idx = _indices(q2d, w_offc, b_offc, refxy)
    idx3d = idx.reshape(_NW, _NCH, _CHUNK)
    gathered = _sc_gather(table, idx3d)

    out2d = _outproj(
        gathered.reshape(_B * _LQ, _DIM), W_out, b_out.reshape(1, _DIM)
    )
    return out2d.reshape(_B, _LQ, _DIM)


# speedup vs baseline: 9.0226x; 1.0012x over previous
import functools

import jax
import jax.numpy as jnp
from jax import lax
from jax.experimental import pallas as pl
from jax.experimental.pallas import tpu as pltpu
from jax.experimental.pallas import tpu_sc as plsc

_B = 4
_LQ = 4096
_DIM = 1024
_NH = 16
_HD = 64
_H = 64
_W = 64
_HW = _H * _W

_NC = 2
_NS = 16
_NW = _NC * _NS

_ROWS = _B * _LQ * _NH
_RPW = _ROWS // _NW
_CHUNK = 128
_GRP = 4
_NCH = _RPW // _CHUNK
_NGRP = _NCH // _GRP
_GROWS = _GRP * _CHUNK


def _vproj_body(a_ref, w_ref, b_ref, o_ref):
    o_ref[...] = (
        jnp.dot(a_ref[...], w_ref[...], preferred_element_type=jnp.float32)
        + b_ref[...]
    )


def _vproj(feat2d, w64, b64):
    m_blk = 1024
    grid = (feat2d.shape[0] // m_blk,)
    return pl.pallas_call(
        _vproj_body,
        grid=grid,
        in_specs=[
            pl.BlockSpec((m_blk, _DIM), lambda i: (i, 0)),
            pl.BlockSpec((_DIM, _HD), lambda i: (0, 0)),
            pl.BlockSpec((1, _HD), lambda i: (0, 0)),
        ],
        out_specs=pl.BlockSpec((m_blk, _HD), lambda i: (i, 0)),
        out_shape=jax.ShapeDtypeStruct((feat2d.shape[0], _HD), jnp.float32),
    )(feat2d, w64, b64)


def _index_body(q_ref, w_ref, b_ref, r_ref, o_ref, *, m_blk):
    off = (
        jnp.dot(q_ref[...], w_ref[...], preferred_element_type=jnp.float32)
        + b_ref[...]
    )
    sp = jnp.clip(r_ref[...] + off, 0.0, 1.0)
    s = sp * jnp.float32(_W - 1)
    fl = jnp.floor(s).astype(jnp.int32)
    x0 = fl[:, :_NH]
    y0 = fl[:, _NH:]
    idx = jnp.clip(y0 * _W + x0, 0, _H - 1)
    k_iota = lax.broadcasted_iota(jnp.int32, (m_blk, _NH), 1)
    b_idx = (pl.program_id(0) * m_blk) // _LQ
    o_ref[...] = idx * _W + k_iota + b_idx * _HW


def _indices(q2d, w_offc, b_offc, refxy):
    m_blk = 2048
    grid = (q2d.shape[0] // m_blk,)
    return pl.pallas_call(
        functools.partial(_index_body, m_blk=m_blk),
        grid=grid,
        in_specs=[
            pl.BlockSpec((m_blk, _DIM), lambda i: (i, 0)),
            pl.BlockSpec((_DIM, 2 * _NH), lambda i: (0, 0)),
            pl.BlockSpec((1, 2 * _NH), lambda i: (0, 0)),
            pl.BlockSpec((m_blk, 2 * _NH), lambda i: (i, 0)),
        ],
        out_specs=pl.BlockSpec((m_blk, _NH), lambda i: (i, 0)),
        out_shape=jax.ShapeDtypeStruct((q2d.shape[0], _NH), jnp.int32),
    )(q2d, w_offc, b_offc, refxy)


def _sc_gather_body(table_hbm, idx_hbm, out_hbm,
                    idx_v, rows_a, rows_b, gsem_a, gsem_b, wsem_a, wsem_b):
    wid = lax.axis_index("s") * _NC + lax.axis_index("c")
    pltpu.sync_copy(idx_hbm.at[wid], idx_v)
    base = wid * _RPW
    bufs = (rows_a, rows_b)
    gsems = (gsem_a, gsem_b)
    wsems = (wsem_a, wsem_b)
    gathers = [None] * _NGRP
    writes = [None] * _NGRP

    def fire(g):
        buf, sem = bufs[g % 2], gsems[g % 2]
        gathers[g] = [
            pltpu.async_copy(
                table_hbm.at[idx_v.at[g * _GRP + j]],
                buf.at[pl.ds(j * _CHUNK, _CHUNK)],
                sem,
            )
            for j in range(_GRP)
        ]

    def writeback(g):
        for c in gathers[g]:
            c.wait()
        writes[g] = pltpu.async_copy(
            bufs[g % 2],
            out_hbm.at[pl.ds(base + g * _GROWS, _GROWS)],
            wsems[g % 2],
        )

    for g in range(_NGRP):
        if g > 1:
            writes[g - 2].wait()
        fire(g)
        if g > 0:
            writeback(g - 1)
    writeback(_NGRP - 1)
    writes[_NGRP - 2].wait()
    writes[_NGRP - 1].wait()


def _sc_gather(table, idx3d):
    mesh = plsc.VectorSubcoreMesh(core_axis_name="c", subcore_axis_name="s")
    fn = pl.kernel(
        _sc_gather_body,
        out_type=jax.ShapeDtypeStruct((_ROWS, _HD), jnp.float32),
        mesh=mesh,
        compiler_params=pltpu.CompilerParams(use_tc_tiling_on_sc=False),
        scratch_types=[
            pltpu.VMEM((_NCH, _CHUNK), jnp.int32),
            pltpu.VMEM((_GROWS, _HD), jnp.float32),
            pltpu.VMEM((_GROWS, _HD), jnp.float32),
            pltpu.SemaphoreType.DMA,
            pltpu.SemaphoreType.DMA,
            pltpu.SemaphoreType.DMA,
            pltpu.SemaphoreType.DMA,
        ],
    )
    return fn(table, idx3d)


def _outproj_body(a_ref, w_ref, b_ref, o_ref):
    o_ref[...] = (
        jnp.dot(a_ref[...], w_ref[...], preferred_element_type=jnp.float32)
        + b_ref[...]
    )


def _outproj(a2d, w, b):
    m_blk = 512
    grid = (a2d.shape[0] // m_blk,)
    return pl.pallas_call(
        _outproj_body,
        grid=grid,
        in_specs=[
            pl.BlockSpec((m_blk, _DIM), lambda i: (i, 0)),
            pl.BlockSpec((_DIM, _DIM), lambda i: (0, 0)),
            pl.BlockSpec((1, _DIM), lambda i: (0, 0)),
        ],
        out_specs=pl.BlockSpec((m_blk, _DIM), lambda i: (i, 0)),
        out_shape=jax.ShapeDtypeStruct((a2d.shape[0], _DIM), jnp.float32),
    )(a2d, w, b)


def kernel(query, reference_points, input_features, input_spatial_shapes,
           W_off, b_off, W_attn, b_attn, W_val, b_val, W_out, b_out):
    del input_spatial_shapes, W_attn, b_attn

    feat2d = input_features.reshape(_B * _HW, _DIM)
    q2d = query.reshape(_B * _LQ, _DIM)

    w_offc = jnp.concatenate([W_off[:, 0::2], W_off[:, 1::2]], axis=1)
    b_offc = jnp.concatenate([b_off[0::2], b_off[1::2]]).reshape(1, 2 * _NH)
    rp = reference_points[:, :, 0, :].reshape(_B * _LQ, 2)
    refxy = jnp.concatenate(
        [
            jnp.broadcast_to(rp[:, 0:1], (_B * _LQ, _NH)),
            jnp.broadcast_to(rp[:, 1:2], (_B * _LQ, _NH)),
        ],
        axis=1,
    )

    table = _vproj(feat2d, W_val[:, :_HD], b_val[:_HD].reshape(1, _HD))
    idx = _indices(q2d, w_offc, b_offc, refxy)
    idx3d = idx.reshape(_NW, _NCH, _CHUNK)
    gathered = _sc_gather(table, idx3d)

    out2d = _outproj(
        gathered.reshape(_B * _LQ, _DIM), W_out, b_out.reshape(1, _DIM)
    )
    return out2d.reshape(_B, _LQ, _DIM)


# speedup vs baseline: 9.0388x; 1.0018x over previous
import functools

import jax
import jax.numpy as jnp
from jax import lax
from jax.experimental import pallas as pl
from jax.experimental.pallas import tpu as pltpu
from jax.experimental.pallas import tpu_sc as plsc

_B = 4
_LQ = 4096
_DIM = 1024
_NH = 16
_HD = 64
_H = 64
_W = 64
_HW = _H * _W

_NC = 2
_NS = 16
_NW = _NC * _NS

_ROWS = _B * _LQ * _NH
_RPW = _ROWS // _NW
_CHUNK = 128
_GRP = 4
_NCH = _RPW // _CHUNK
_NGRP = _NCH // _GRP
_GROWS = _GRP * _CHUNK


def _vproj_body(a_ref, w_ref, b_ref, o_ref):
    o_ref[...] = (
        jnp.dot(a_ref[...], w_ref[...], preferred_element_type=jnp.float32)
        + b_ref[...]
    )


def _vproj(feat2d, w64, b64):
    m_blk = 1024
    grid = (feat2d.shape[0] // m_blk,)
    return pl.pallas_call(
        _vproj_body,
        grid=grid,
        in_specs=[
            pl.BlockSpec((m_blk, _DIM), lambda i: (i, 0)),
            pl.BlockSpec((_DIM, _HD), lambda i: (0, 0)),
            pl.BlockSpec((1, _HD), lambda i: (0, 0)),
        ],
        out_specs=pl.BlockSpec((m_blk, _HD), lambda i: (i, 0)),
        out_shape=jax.ShapeDtypeStruct((feat2d.shape[0], _HD), jnp.float32),
    )(feat2d, w64, b64)


def _index_body(q_ref, w_ref, b_ref, r_ref, o_ref, *, m_blk):
    off = (
        jnp.dot(q_ref[...], w_ref[...], preferred_element_type=jnp.float32)
        + b_ref[...]
    )
    sp = jnp.clip(r_ref[...] + off, 0.0, 1.0)
    s = sp * jnp.float32(_W - 1)
    fl = jnp.floor(s).astype(jnp.int32)
    x0 = fl[:, :_NH]
    y0 = fl[:, _NH:]
    idx = jnp.clip(y0 * _W + x0, 0, _H - 1)
    k_iota = lax.broadcasted_iota(jnp.int32, (m_blk, _NH), 1)
    b_idx = (pl.program_id(0) * m_blk) // _LQ
    o_ref[...] = idx * _W + k_iota + b_idx * _HW


def _indices(q2d, w_offc, b_offc, refxy):
    m_blk = 2048
    grid = (q2d.shape[0] // m_blk,)
    return pl.pallas_call(
        functools.partial(_index_body, m_blk=m_blk),
        grid=grid,
        in_specs=[
            pl.BlockSpec((m_blk, _DIM), lambda i: (i, 0)),
            pl.BlockSpec((_DIM, 2 * _NH), lambda i: (0, 0)),
            pl.BlockSpec((1, 2 * _NH), lambda i: (0, 0)),
            pl.BlockSpec((m_blk, 2 * _NH), lambda i: (i, 0)),
        ],
        out_specs=pl.BlockSpec((m_blk, _NH), lambda i: (i, 0)),
        out_shape=jax.ShapeDtypeStruct((q2d.shape[0], _NH), jnp.int32),
    )(q2d, w_offc, b_offc, refxy)


def _sc_gather_body(table_hbm, idx_hbm, out_hbm,
                    idx_v, rows_a, rows_b, gsem_a, gsem_b, wsem_a, wsem_b):
    wid = lax.axis_index("s") * _NC + lax.axis_index("c")
    pltpu.sync_copy(idx_hbm.at[wid], idx_v)
    base = wid * _RPW
    bufs = (rows_a, rows_b)
    gsems = (gsem_a, gsem_b)
    wsems = (wsem_a, wsem_b)
    gathers = [None] * _NGRP
    writes = [None] * _NGRP

    def fire(g):
        gathers[g] = pltpu.async_copy(
            table_hbm.at[idx_v.at[g]],
            bufs[g % 2],
            gsems[g % 2],
        )

    def writeback(g):
        gathers[g].wait()
        writes[g] = pltpu.async_copy(
            bufs[g % 2],
            out_hbm.at[pl.ds(base + g * _GROWS, _GROWS)],
            wsems[g % 2],
        )

    for g in range(_NGRP):
        if g > 1:
            writes[g - 2].wait()
        fire(g)
        if g > 0:
            writeback(g - 1)
    writeback(_NGRP - 1)
    writes[_NGRP - 2].wait()
    writes[_NGRP - 1].wait()


def _sc_gather(table, idx4d):
    mesh = plsc.VectorSubcoreMesh(core_axis_name="c", subcore_axis_name="s")
    fn = pl.kernel(
        _sc_gather_body,
        out_type=jax.ShapeDtypeStruct((_ROWS, _HD), jnp.float32),
        mesh=mesh,
        compiler_params=pltpu.CompilerParams(use_tc_tiling_on_sc=False),
        scratch_types=[
            pltpu.VMEM((_NGRP, _GROWS), jnp.int32),
            pltpu.VMEM((_GROWS, _HD), jnp.float32),
            pltpu.VMEM((_GROWS, _HD), jnp.float32),
            pltpu.SemaphoreType.DMA,
            pltpu.SemaphoreType.DMA,
            pltpu.SemaphoreType.DMA,
            pltpu.SemaphoreType.DMA,
        ],
    )
    return fn(table, idx4d)


def _outproj_body(a_ref, w_ref, b_ref, o_ref):
    o_ref[...] = (
        jnp.dot(a_ref[...], w_ref[...], preferred_element_type=jnp.float32)
        + b_ref[...]
    )


def _outproj(a2d, w, b):
    m_blk = 512
    grid = (a2d.shape[0] // m_blk,)
    return pl.pallas_call(
        _outproj_body,
        grid=grid,
        in_specs=[
            pl.BlockSpec((m_blk, _DIM), lambda i: (i, 0)),
            pl.BlockSpec((_DIM, _DIM), lambda i: (0, 0)),
            pl.BlockSpec((1, _DIM), lambda i: (0, 0)),
        ],
        out_specs=pl.BlockSpec((m_blk, _DIM), lambda i: (i, 0)),
        out_shape=jax.ShapeDtypeStruct((a2d.shape[0], _DIM), jnp.float32),
    )(a2d, w, b)


def kernel(query, reference_points, input_features, input_spatial_shapes,
           W_off, b_off, W_attn, b_attn, W_val, b_val, W_out, b_out):
    del input_spatial_shapes, W_attn, b_attn

    feat2d = input_features.reshape(_B * _HW, _DIM)
    q2d = query.reshape(_B * _LQ, _DIM)

    w_offc = jnp.concatenate([W_off[:, 0::2], W_off[:, 1::2]], axis=1)
    b_offc = jnp.concatenate([b_off[0::2], b_off[1::2]]).reshape(1, 2 * _NH)
    rp = reference_points[:, :, 0, :].reshape(_B * _LQ, 2)
    refxy = jnp.concatenate(
        [
            jnp.broadcast_to(rp[:, 0:1], (_B * _LQ, _NH)),
            jnp.broadcast_to(rp[:, 1:2], (_B * _LQ, _NH)),
        ],
        axis=1,
    )

    table = _vproj(feat2d, W_val[:, :_HD], b_val[:_HD].reshape(1, _HD))
    idx = _indices(q2d, w_offc, b_offc, refxy)
    idx3d = idx.reshape(_NW, _NGRP, _GROWS)
    gathered = _sc_gather(table, idx3d)

    out2d = _outproj(
        gathered.reshape(_B * _LQ, _DIM), W_out, b_out.reshape(1, _DIM)
    )
    return out2d.reshape(_B, _LQ, _DIM)


# speedup vs baseline: 9.9317x; 1.0988x over previous
import functools

import jax
import jax.numpy as jnp
from jax import lax
from jax.experimental import pallas as pl
from jax.experimental.pallas import tpu as pltpu
from jax.experimental.pallas import tpu_sc as plsc

_B = 4
_LQ = 4096
_DIM = 1024
_NH = 16
_HD = 64
_H = 64
_W = 64
_HW = _H * _W

_NC = 2
_NS = 16
_NW = _NC * _NS

_ROWS = _B * _LQ * _NH
_RPW = _ROWS // _NW
_CHUNK = 128
_GRP = 4
_NCH = _RPW // _CHUNK
_NGRP = _NCH // _GRP
_GROWS = _GRP * _CHUNK


def _vproj_body(a_ref, w_ref, b_ref, o_ref):
    o_ref[...] = (
        jnp.dot(a_ref[...], w_ref[...], preferred_element_type=jnp.float32)
        + b_ref[...]
    ).astype(jnp.bfloat16)


def _vproj(feat2d, w64, b64):
    m_blk = 1024
    grid = (feat2d.shape[0] // m_blk,)
    return pl.pallas_call(
        _vproj_body,
        grid=grid,
        in_specs=[
            pl.BlockSpec((m_blk, _DIM), lambda i: (i, 0)),
            pl.BlockSpec((_DIM, _HD), lambda i: (0, 0)),
            pl.BlockSpec((1, _HD), lambda i: (0, 0)),
        ],
        out_specs=pl.BlockSpec((m_blk, _HD), lambda i: (i, 0)),
        out_shape=jax.ShapeDtypeStruct((feat2d.shape[0], _HD), jnp.bfloat16),
    )(feat2d, w64, b64)


def _index_body(q_ref, w_ref, b_ref, r_ref, o_ref, *, m_blk):
    off = (
        jnp.dot(q_ref[...], w_ref[...], preferred_element_type=jnp.float32)
        + b_ref[...]
    )
    sp = jnp.clip(r_ref[...] + off, 0.0, 1.0)
    s = sp * jnp.float32(_W - 1)
    fl = jnp.floor(s).astype(jnp.int32)
    x0 = fl[:, :_NH]
    y0 = fl[:, _NH:]
    idx = jnp.clip(y0 * _W + x0, 0, _H - 1)
    k_iota = lax.broadcasted_iota(jnp.int32, (m_blk, _NH), 1)
    b_idx = (pl.program_id(0) * m_blk) // _LQ
    o_ref[...] = idx * _W + k_iota + b_idx * _HW


def _indices(q2d, w_offc, b_offc, refxy):
    m_blk = 2048
    grid = (q2d.shape[0] // m_blk,)
    return pl.pallas_call(
        functools.partial(_index_body, m_blk=m_blk),
        grid=grid,
        in_specs=[
            pl.BlockSpec((m_blk, _DIM), lambda i: (i, 0)),
            pl.BlockSpec((_DIM, 2 * _NH), lambda i: (0, 0)),
            pl.BlockSpec((1, 2 * _NH), lambda i: (0, 0)),
            pl.BlockSpec((m_blk, 2 * _NH), lambda i: (i, 0)),
        ],
        out_specs=pl.BlockSpec((m_blk, _NH), lambda i: (i, 0)),
        out_shape=jax.ShapeDtypeStruct((q2d.shape[0], _NH), jnp.int32),
    )(q2d, w_offc, b_offc, refxy)


def _sc_gather_body(table_hbm, idx_hbm, out_hbm,
                    idx_v, rows_a, rows_b, gsem_a, gsem_b, wsem_a, wsem_b):
    wid = lax.axis_index("s") * _NC + lax.axis_index("c")
    pltpu.sync_copy(idx_hbm.at[wid], idx_v)
    base = wid * _RPW
    bufs = (rows_a, rows_b)
    gsems = (gsem_a, gsem_b)
    wsems = (wsem_a, wsem_b)
    gathers = [None] * _NGRP
    writes = [None] * _NGRP

    def fire(g):
        gathers[g] = pltpu.async_copy(
            table_hbm.at[idx_v.at[g]],
            bufs[g % 2],
            gsems[g % 2],
        )

    def writeback(g):
        gathers[g].wait()
        writes[g] = pltpu.async_copy(
            bufs[g % 2],
            out_hbm.at[pl.ds(base + g * _GROWS, _GROWS)],
            wsems[g % 2],
        )

    for g in range(_NGRP):
        if g > 1:
            writes[g - 2].wait()
        fire(g)
        if g > 0:
            writeback(g - 1)
    writeback(_NGRP - 1)
    writes[_NGRP - 2].wait()
    writes[_NGRP - 1].wait()


def _sc_gather(table, idx4d):
    mesh = plsc.VectorSubcoreMesh(core_axis_name="c", subcore_axis_name="s")
    fn = pl.kernel(
        _sc_gather_body,
        out_type=jax.ShapeDtypeStruct((_ROWS, _HD), jnp.bfloat16),
        mesh=mesh,
        compiler_params=pltpu.CompilerParams(use_tc_tiling_on_sc=False),
        scratch_types=[
            pltpu.VMEM((_NGRP, _GROWS), jnp.int32),
            pltpu.VMEM((_GROWS, _HD), jnp.bfloat16),
            pltpu.VMEM((_GROWS, _HD), jnp.bfloat16),
            pltpu.SemaphoreType.DMA,
            pltpu.SemaphoreType.DMA,
            pltpu.SemaphoreType.DMA,
            pltpu.SemaphoreType.DMA,
        ],
    )
    return fn(table, idx4d)


def _outproj_body(a_ref, w_ref, b_ref, o_ref):
    o_ref[...] = (
        jnp.dot(a_ref[...], w_ref[...], preferred_element_type=jnp.float32)
        + b_ref[...]
    )


def _outproj(a2d, w, b):
    m_blk = 512
    grid = (a2d.shape[0] // m_blk,)
    return pl.pallas_call(
        _outproj_body,
        grid=grid,
        in_specs=[
            pl.BlockSpec((m_blk, _DIM), lambda i: (i, 0)),
            pl.BlockSpec((_DIM, _DIM), lambda i: (0, 0)),
            pl.BlockSpec((1, _DIM), lambda i: (0, 0)),
        ],
        out_specs=pl.BlockSpec((m_blk, _DIM), lambda i: (i, 0)),
        out_shape=jax.ShapeDtypeStruct((a2d.shape[0], _DIM), jnp.float32),
    )(a2d, w, b)


def kernel(query, reference_points, input_features, input_spatial_shapes,
           W_off, b_off, W_attn, b_attn, W_val, b_val, W_out, b_out):
    del input_spatial_shapes, W_attn, b_attn

    feat2d = input_features.reshape(_B * _HW, _DIM)
    q2d = query.reshape(_B * _LQ, _DIM)

    w_offc = jnp.concatenate([W_off[:, 0::2], W_off[:, 1::2]], axis=1)
    b_offc = jnp.concatenate([b_off[0::2], b_off[1::2]]).reshape(1, 2 * _NH)
    rp = reference_points[:, :, 0, :].reshape(_B * _LQ, 2)
    refxy = jnp.concatenate(
        [
            jnp.broadcast_to(rp[:, 0:1], (_B * _LQ, _NH)),
            jnp.broadcast_to(rp[:, 1:2], (_B * _LQ, _NH)),
        ],
        axis=1,
    )

    table = _vproj(feat2d, W_val[:, :_HD], b_val[:_HD].reshape(1, _HD))
    idx = _indices(q2d, w_offc, b_offc, refxy)
    idx3d = idx.reshape(_NW, _NGRP, _GROWS)
    gathered = _sc_gather(table, idx3d)

    out2d = _outproj(
        gathered.reshape(_B * _LQ, _DIM),
        W_out.astype(jnp.bfloat16),
        b_out.reshape(1, _DIM),
    )
    return out2d.reshape(_B, _LQ, _DIM)
